# Initial kernel scaffold; baseline (speedup 1.0000x reference)
import jax
import jax.numpy as jnp
from jax.experimental import pallas as pl


def kernel(features, edge_index, W1, b1, W2, b2):
    raise NotImplementedError("write your pallas kernel here")


# speedup vs baseline: 3.9564x; 3.9564x over previous
import functools

import jax
import jax.numpy as jnp
from jax import lax
from jax.experimental import pallas as pl
from jax.experimental.pallas import tpu as pltpu
from jax.experimental.pallas import tpu_sc as plsc

N = 10000
E = 320000
D = 128

NC = 2
NS = 16
NW = NC * NS
EPT = E // NW
NP = 10240
RPS = NP // NS
CHUNK = 80
NCHUNK = EPT // CHUNK
CW = 16

_mesh = plsc.VectorSubcoreMesh(core_axis_name="c", subcore_axis_name="s")
_sc_params = pltpu.CompilerParams(use_tc_tiling_on_sc=False)


def _degree_body(src_hbm, dst_hbm, zeros_hbm, ones_hbm,
                 scnt_hbm, dcnt_hbm, idx_v, ones_v, scnt_sh, dcnt_sh):
    c = lax.axis_index("c")
    s = lax.axis_index("s")
    wid = c * NS + s
    pltpu.sync_copy(ones_hbm, ones_v)
    pltpu.sync_copy(zeros_hbm, scnt_sh.at[pl.ds(s * RPS, RPS)])
    pltpu.sync_copy(zeros_hbm, dcnt_sh.at[pl.ds(s * RPS, RPS)])
    plsc.subcore_barrier()
    base0 = wid * EPT

    @pl.loop(0, NCHUNK)
    def _(i):
        base = base0 + i * CHUNK
        pltpu.sync_copy(src_hbm.at[pl.ds(base, CHUNK)], idx_v)
        pltpu.sync_copy(ones_v, scnt_sh.at[idx_v], add=True)
        pltpu.sync_copy(dst_hbm.at[pl.ds(base, CHUNK)], idx_v)
        pltpu.sync_copy(ones_v, dcnt_sh.at[idx_v], add=True)

    plsc.subcore_barrier()
    rows = pl.ds(s * RPS, RPS)
    pltpu.sync_copy(scnt_sh.at[rows], scnt_hbm.at[c, rows])
    pltpu.sync_copy(dcnt_sh.at[rows], dcnt_hbm.at[c, rows])


def _degrees(src, dst):
    zeros = jnp.zeros((RPS, CW), jnp.float32)
    ones = jnp.ones((CHUNK, CW), jnp.float32)
    f = pl.kernel(
        _degree_body,
        out_type=(jax.ShapeDtypeStruct((NC, NP, CW), jnp.float32),
                  jax.ShapeDtypeStruct((NC, NP, CW), jnp.float32)),
        mesh=_mesh,
        scratch_types=[
            pltpu.VMEM((CHUNK,), jnp.int32),
            pltpu.VMEM((CHUNK, CW), jnp.float32),
            pltpu.VMEM_SHARED((NP, CW), jnp.float32),
            pltpu.VMEM_SHARED((NP, CW), jnp.float32),
        ],
        compiler_params=_sc_params,
    )
    return f(src, dst, zeros, ones)


def _seg_body(h_hbm, src_hbm, dst_hbm, zeros_hbm, out_hbm, idx_v, rows_v, acc_sh):
    c = lax.axis_index("c")
    s = lax.axis_index("s")
    wid = c * NS + s
    pltpu.sync_copy(zeros_hbm, acc_sh.at[pl.ds(s * RPS, RPS)])
    plsc.subcore_barrier()
    base0 = wid * EPT

    @pl.loop(0, NCHUNK)
    def _(i):
        base = base0 + i * CHUNK
        pltpu.sync_copy(src_hbm.at[pl.ds(base, CHUNK)], idx_v)
        pltpu.sync_copy(h_hbm.at[idx_v], rows_v)
        pltpu.sync_copy(dst_hbm.at[pl.ds(base, CHUNK)], idx_v)
        pltpu.sync_copy(rows_v, acc_sh.at[idx_v], add=True)

    plsc.subcore_barrier()
    rows = pl.ds(s * RPS, RPS)
    pltpu.sync_copy(acc_sh.at[rows], out_hbm.at[c, rows])


def _seg_sum(h, src, dst):
    zeros = jnp.zeros((RPS, D), jnp.float32)
    f = pl.kernel(
        _seg_body,
        out_type=jax.ShapeDtypeStruct((NC, NP, D), jnp.float32),
        mesh=_mesh,
        scratch_types=[
            pltpu.VMEM((CHUNK,), jnp.int32),
            pltpu.VMEM((CHUNK, D), jnp.float32),
            pltpu.VMEM_SHARED((NP, D), jnp.float32),
        ],
        compiler_params=_sc_params,
    )
    return f(h, src, dst, zeros)


RB = 400
GRID = N // RB


def _mm_body(x_ref, w_ref, o_ref):
    o_ref[...] = jnp.dot(x_ref[...], w_ref[...],
                         preferred_element_type=jnp.float32,
                         precision=lax.Precision.HIGHEST)


def _matmul(x, w):
    return pl.pallas_call(
        _mm_body,
        grid=(GRID,),
        in_specs=[pl.BlockSpec((RB, D), lambda i: (i, 0)),
                  pl.BlockSpec((D, D), lambda i: (0, 0))],
        out_specs=pl.BlockSpec((RB, D), lambda i: (i, 0)),
        out_shape=jax.ShapeDtypeStruct((N, D), jnp.float32),
    )(x, w)


def _norm_scale_body(scnt_ref, dcnt_ref, h_ref, h_out, onorm_out, inorm_out):
    sdeg = scnt_ref[0, :, 0:1] + scnt_ref[1, :, 0:1]
    ddeg = dcnt_ref[0, :, 0:1] + dcnt_ref[1, :, 0:1]
    onorm = lax.rsqrt(jnp.maximum(sdeg, 1.0))
    inorm = lax.rsqrt(jnp.maximum(ddeg, 1.0))
    onorm_out[...] = onorm
    inorm_out[...] = inorm
    h_out[...] = h_ref[...] * onorm


def _norm_scale(scnt, dcnt, h_raw):
    cnt_spec = pl.BlockSpec((NC, RB, CW), lambda i: (0, i, 0))
    return pl.pallas_call(
        _norm_scale_body,
        grid=(GRID,),
        in_specs=[cnt_spec, cnt_spec, pl.BlockSpec((RB, D), lambda i: (i, 0))],
        out_specs=[pl.BlockSpec((RB, D), lambda i: (i, 0)),
                   pl.BlockSpec((RB, 1), lambda i: (i, 0)),
                   pl.BlockSpec((RB, 1), lambda i: (i, 0))],
        out_shape=[jax.ShapeDtypeStruct((N, D), jnp.float32),
                   jax.ShapeDtypeStruct((N, 1), jnp.float32),
                   jax.ShapeDtypeStruct((N, 1), jnp.float32)],
    )(scnt, dcnt, h_raw)


def _mid_body(p_ref, inorm_ref, onorm_ref, b1_ref, w2_ref, h2_out):
    x = (p_ref[0] + p_ref[1]) * inorm_ref[...] + b1_ref[...]
    x = jnp.maximum(x, 0.0) * onorm_ref[...]
    h2_out[...] = jnp.dot(x, w2_ref[...],
                          preferred_element_type=jnp.float32,
                          precision=lax.Precision.HIGHEST)


def _mid(p, inorm, onorm, b1, w2):
    return pl.pallas_call(
        _mid_body,
        grid=(GRID,),
        in_specs=[pl.BlockSpec((NC, RB, D), lambda i: (0, i, 0)),
                  pl.BlockSpec((RB, 1), lambda i: (i, 0)),
                  pl.BlockSpec((RB, 1), lambda i: (i, 0)),
                  pl.BlockSpec((1, D), lambda i: (0, 0)),
                  pl.BlockSpec((D, D), lambda i: (0, 0))],
        out_specs=pl.BlockSpec((RB, D), lambda i: (i, 0)),
        out_shape=jax.ShapeDtypeStruct((N, D), jnp.float32),
    )(p, inorm, onorm, b1, w2)


def _final_body(q_ref, inorm_ref, b2_ref, o_ref):
    o_ref[...] = (q_ref[0] + q_ref[1]) * inorm_ref[...] + b2_ref[...]


def _final(q, inorm, b2):
    return pl.pallas_call(
        _final_body,
        grid=(GRID,),
        in_specs=[pl.BlockSpec((NC, RB, D), lambda i: (0, i, 0)),
                  pl.BlockSpec((RB, 1), lambda i: (i, 0)),
                  pl.BlockSpec((1, D), lambda i: (0, 0))],
        out_specs=pl.BlockSpec((RB, D), lambda i: (i, 0)),
        out_shape=jax.ShapeDtypeStruct((N, D), jnp.float32),
    )(q, inorm, b2)


def kernel(features, edge_index, W1, b1, W2, b2):
    src = edge_index[0]
    dst = edge_index[1]
    scnt, dcnt = _degrees(src, dst)
    h1_raw = _matmul(features, W1)
    h1, onorm, inorm = _norm_scale(scnt, dcnt, h1_raw)
    p = _seg_sum(h1, src, dst)
    h2 = _mid(p, inorm, onorm, b1.reshape(1, D), W2)
    q = _seg_sum(h2, src, dst)
    return _final(q, inorm, b2.reshape(1, D))


# speedup vs baseline: 9.2220x; 2.3309x over previous
import functools

import jax
import jax.numpy as jnp
from jax import lax
from jax.experimental import pallas as pl
from jax.experimental.pallas import tpu as pltpu
from jax.experimental.pallas import tpu_sc as plsc

N = 10000
E = 320000
D = 128

NC = 2
NS = 16
NW = NC * NS
EPT = E // NW
NP = 10240
RPS = NP // NS
CHUNK = 100
NCHUNK = EPT // CHUNK
CW = 16

_mesh = plsc.VectorSubcoreMesh(core_axis_name="c", subcore_axis_name="s")
_sc_params = pltpu.CompilerParams(use_tc_tiling_on_sc=False)


def _degree_body(src2_hbm, dst2_hbm, zeros_hbm, ones_hbm,
                 scnt_hbm, dcnt_hbm, sidx_v, didx_v, ones_v,
                 scnt_sh, dcnt_sh, ssem, dsem):
    c = lax.axis_index("c")
    s = lax.axis_index("s")
    wid = c * NS + s
    row0 = wid * NCHUNK
    pltpu.sync_copy(ones_hbm, ones_v)
    pltpu.sync_copy(src2_hbm.at[pl.ds(row0, NCHUNK)], sidx_v)
    pltpu.sync_copy(dst2_hbm.at[pl.ds(row0, NCHUNK)], didx_v)
    pltpu.sync_copy(zeros_hbm, scnt_sh.at[pl.ds(s * RPS, RPS)])
    pltpu.sync_copy(zeros_hbm, dcnt_sh.at[pl.ds(s * RPS, RPS)])
    plsc.subcore_barrier()

    @pl.loop(0, NCHUNK)
    def _(i):
        a = pltpu.async_copy(ones_v, scnt_sh.at[sidx_v.at[i]], ssem, add=True)
        b = pltpu.async_copy(ones_v, dcnt_sh.at[didx_v.at[i]], dsem, add=True)
        a.wait()
        b.wait()

    plsc.subcore_barrier()
    rows = pl.ds(s * RPS, RPS)
    pltpu.sync_copy(scnt_sh.at[rows], scnt_hbm.at[c, rows])
    pltpu.sync_copy(dcnt_sh.at[rows], dcnt_hbm.at[c, rows])


def _degrees(src2, dst2):
    zeros = jnp.zeros((RPS, CW), jnp.float32)
    ones = jnp.ones((CHUNK, CW), jnp.float32)
    f = pl.kernel(
        _degree_body,
        out_type=(jax.ShapeDtypeStruct((NC, NP, CW), jnp.float32),
                  jax.ShapeDtypeStruct((NC, NP, CW), jnp.float32)),
        mesh=_mesh,
        scratch_types=[
            pltpu.VMEM((NCHUNK, CHUNK), jnp.int32),
            pltpu.VMEM((NCHUNK, CHUNK), jnp.int32),
            pltpu.VMEM((CHUNK, CW), jnp.float32),
            pltpu.VMEM_SHARED((NP, CW), jnp.float32),
            pltpu.VMEM_SHARED((NP, CW), jnp.float32),
            pltpu.SemaphoreType.DMA,
            pltpu.SemaphoreType.DMA,
        ],
        compiler_params=_sc_params,
    )
    return f(src2, dst2, zeros, ones)


def _seg_body(h_hbm, src2_hbm, dst2_hbm, zeros_hbm, out_hbm,
              sidx_v, didx_v, rows0_v, rows1_v, acc_sh, gsem0, gsem1):
    c = lax.axis_index("c")
    s = lax.axis_index("s")
    wid = c * NS + s
    row0 = wid * NCHUNK
    pltpu.sync_copy(src2_hbm.at[pl.ds(row0, NCHUNK)], sidx_v)
    pltpu.sync_copy(dst2_hbm.at[pl.ds(row0, NCHUNK)], didx_v)
    pltpu.sync_copy(zeros_hbm, acc_sh.at[pl.ds(s * RPS, RPS)])
    plsc.subcore_barrier()

    pltpu.async_copy(h_hbm.at[sidx_v.at[0]], rows0_v, gsem0)

    @pl.loop(0, NCHUNK // 2)
    def _(g2):
        a = 2 * g2
        pltpu.async_copy(h_hbm.at[sidx_v.at[a + 1]], rows1_v, gsem1)
        pltpu.make_async_copy(h_hbm.at[sidx_v.at[a]], rows0_v, gsem0).wait()
        pltpu.sync_copy(rows0_v, acc_sh.at[didx_v.at[a]], add=True)
        nxt = jnp.minimum(a + 2, NCHUNK - 1)
        pltpu.async_copy(h_hbm.at[sidx_v.at[nxt]], rows0_v, gsem0)
        pltpu.make_async_copy(h_hbm.at[sidx_v.at[a + 1]], rows1_v, gsem1).wait()
        pltpu.sync_copy(rows1_v, acc_sh.at[didx_v.at[a + 1]], add=True)

    pltpu.make_async_copy(h_hbm.at[sidx_v.at[0]], rows0_v, gsem0).wait()

    plsc.subcore_barrier()
    rows = pl.ds(s * RPS, RPS)
    pltpu.sync_copy(acc_sh.at[rows], out_hbm.at[c, rows])


def _seg_sum(h, src2, dst2):
    zeros = jnp.zeros((RPS, D), jnp.float32)
    f = pl.kernel(
        _seg_body,
        out_type=jax.ShapeDtypeStruct((NC, NP, D), jnp.float32),
        mesh=_mesh,
        scratch_types=[
            pltpu.VMEM((NCHUNK, CHUNK), jnp.int32),
            pltpu.VMEM((NCHUNK, CHUNK), jnp.int32),
            pltpu.VMEM((CHUNK, D), jnp.float32),
            pltpu.VMEM((CHUNK, D), jnp.float32),
            pltpu.VMEM_SHARED((NP, D), jnp.float32),
            pltpu.SemaphoreType.DMA,
            pltpu.SemaphoreType.DMA,
        ],
        compiler_params=_sc_params,
    )
    return f(h, src2, dst2, zeros)


RB = 400
GRID = N // RB


def _mm_body(x_ref, w_ref, o_ref):
    o_ref[...] = jnp.dot(x_ref[...], w_ref[...],
                         preferred_element_type=jnp.float32,
                         precision=lax.Precision.HIGHEST)


def _matmul(x, w):
    return pl.pallas_call(
        _mm_body,
        grid=(GRID,),
        in_specs=[pl.BlockSpec((RB, D), lambda i: (i, 0)),
                  pl.BlockSpec((D, D), lambda i: (0, 0))],
        out_specs=pl.BlockSpec((RB, D), lambda i: (i, 0)),
        out_shape=jax.ShapeDtypeStruct((N, D), jnp.float32),
    )(x, w)


def _norm_scale_body(scnt_ref, dcnt_ref, h_ref, h_out, onorm_out, inorm_out):
    sdeg = scnt_ref[0, :, 0:1] + scnt_ref[1, :, 0:1]
    ddeg = dcnt_ref[0, :, 0:1] + dcnt_ref[1, :, 0:1]
    onorm = lax.rsqrt(jnp.maximum(sdeg, 1.0))
    inorm = lax.rsqrt(jnp.maximum(ddeg, 1.0))
    onorm_out[...] = onorm
    inorm_out[...] = inorm
    h_out[...] = h_ref[...] * onorm


def _norm_scale(scnt, dcnt, h_raw):
    cnt_spec = pl.BlockSpec((NC, RB, CW), lambda i: (0, i, 0))
    return pl.pallas_call(
        _norm_scale_body,
        grid=(GRID,),
        in_specs=[cnt_spec, cnt_spec, pl.BlockSpec((RB, D), lambda i: (i, 0))],
        out_specs=[pl.BlockSpec((RB, D), lambda i: (i, 0)),
                   pl.BlockSpec((RB, 1), lambda i: (i, 0)),
                   pl.BlockSpec((RB, 1), lambda i: (i, 0))],
        out_shape=[jax.ShapeDtypeStruct((N, D), jnp.float32),
                   jax.ShapeDtypeStruct((N, 1), jnp.float32),
                   jax.ShapeDtypeStruct((N, 1), jnp.float32)],
    )(scnt, dcnt, h_raw)


def _mid_body(p_ref, inorm_ref, onorm_ref, b1_ref, w2_ref, h2_out):
    x = (p_ref[0] + p_ref[1]) * inorm_ref[...] + b1_ref[...]
    x = jnp.maximum(x, 0.0) * onorm_ref[...]
    h2_out[...] = jnp.dot(x, w2_ref[...],
                          preferred_element_type=jnp.float32,
                          precision=lax.Precision.HIGHEST)


def _mid(p, inorm, onorm, b1, w2):
    return pl.pallas_call(
        _mid_body,
        grid=(GRID,),
        in_specs=[pl.BlockSpec((NC, RB, D), lambda i: (0, i, 0)),
                  pl.BlockSpec((RB, 1), lambda i: (i, 0)),
                  pl.BlockSpec((RB, 1), lambda i: (i, 0)),
                  pl.BlockSpec((1, D), lambda i: (0, 0)),
                  pl.BlockSpec((D, D), lambda i: (0, 0))],
        out_specs=pl.BlockSpec((RB, D), lambda i: (i, 0)),
        out_shape=jax.ShapeDtypeStruct((N, D), jnp.float32),
    )(p, inorm, onorm, b1, w2)


def _final_body(q_ref, inorm_ref, b2_ref, o_ref):
    o_ref[...] = (q_ref[0] + q_ref[1]) * inorm_ref[...] + b2_ref[...]


def _final(q, inorm, b2):
    return pl.pallas_call(
        _final_body,
        grid=(GRID,),
        in_specs=[pl.BlockSpec((NC, RB, D), lambda i: (0, i, 0)),
                  pl.BlockSpec((RB, 1), lambda i: (i, 0)),
                  pl.BlockSpec((1, D), lambda i: (0, 0))],
        out_specs=pl.BlockSpec((RB, D), lambda i: (i, 0)),
        out_shape=jax.ShapeDtypeStruct((N, D), jnp.float32),
    )(q, inorm, b2)


def kernel(features, edge_index, W1, b1, W2, b2):
    src2 = edge_index[0].reshape(E // CHUNK, CHUNK)
    dst2 = edge_index[1].reshape(E // CHUNK, CHUNK)
    scnt, dcnt = _degrees(src2, dst2)
    h1_raw = _matmul(features, W1)
    h1, onorm, inorm = _norm_scale(scnt, dcnt, h1_raw)
    p = _seg_sum(h1, src2, dst2)
    h2 = _mid(p, inorm, onorm, b1.reshape(1, D), W2)
    q = _seg_sum(h2, src2, dst2)
    return _final(q, inorm, b2.reshape(1, D))


# speedup vs baseline: 9.8843x; 1.0718x over previous
import functools

import jax
import jax.numpy as jnp
from jax import lax
from jax.experimental import pallas as pl
from jax.experimental.pallas import tpu as pltpu
from jax.experimental.pallas import tpu_sc as plsc

N = 10000
E = 320000
D = 128

NC = 2
NS = 16
NW = NC * NS
EPT = E // NW
NP = 10240
RPS = NP // NS
CHUNK = 100
NCHUNK = EPT // CHUNK
CW = 16

_mesh = plsc.VectorSubcoreMesh(core_axis_name="c", subcore_axis_name="s")
_sc_params = pltpu.CompilerParams(use_tc_tiling_on_sc=False)


def _degree_body(src2_hbm, dst2_hbm, zeros_hbm, ones_hbm,
                 scnt_hbm, dcnt_hbm, sidx_v, didx_v, ones_v,
                 scnt_sh, dcnt_sh, ssem, dsem):
    c = lax.axis_index("c")
    s = lax.axis_index("s")
    wid = c * NS + s
    row0 = wid * NCHUNK
    pltpu.sync_copy(ones_hbm, ones_v)
    pltpu.sync_copy(src2_hbm.at[pl.ds(row0, NCHUNK)], sidx_v)
    pltpu.sync_copy(dst2_hbm.at[pl.ds(row0, NCHUNK)], didx_v)
    pltpu.sync_copy(zeros_hbm, scnt_sh.at[pl.ds(s * RPS, RPS)])
    pltpu.sync_copy(zeros_hbm, dcnt_sh.at[pl.ds(s * RPS, RPS)])
    plsc.subcore_barrier()

    @pl.loop(0, NCHUNK)
    def _(i):
        a = pltpu.async_copy(ones_v, scnt_sh.at[sidx_v.at[i]], ssem, add=True)
        b = pltpu.async_copy(ones_v, dcnt_sh.at[didx_v.at[i]], dsem, add=True)
        a.wait()
        b.wait()

    plsc.subcore_barrier()
    rows = pl.ds(s * RPS, RPS)
    pltpu.sync_copy(scnt_sh.at[rows], scnt_hbm.at[c, rows])
    pltpu.sync_copy(dcnt_sh.at[rows], dcnt_hbm.at[c, rows])


def _degrees(src2, dst2):
    zeros = jnp.zeros((RPS, CW), jnp.float32)
    ones = jnp.ones((CHUNK, CW), jnp.float32)
    f = pl.kernel(
        _degree_body,
        out_type=(jax.ShapeDtypeStruct((NC, NP, CW), jnp.float32),
                  jax.ShapeDtypeStruct((NC, NP, CW), jnp.float32)),
        mesh=_mesh,
        scratch_types=[
            pltpu.VMEM((NCHUNK, CHUNK), jnp.int32),
            pltpu.VMEM((NCHUNK, CHUNK), jnp.int32),
            pltpu.VMEM((CHUNK, CW), jnp.float32),
            pltpu.VMEM_SHARED((NP, CW), jnp.float32),
            pltpu.VMEM_SHARED((NP, CW), jnp.float32),
            pltpu.SemaphoreType.DMA,
            pltpu.SemaphoreType.DMA,
        ],
        compiler_params=_sc_params,
    )
    return f(src2, dst2, zeros, ones)


def _seg_body(h_hbm, src2_hbm, dst2_hbm, zeros_hbm, out_hbm,
              sidx_v, didx_v, rows0_v, rows1_v, acc_sh, gsem0, gsem1):
    c = lax.axis_index("c")
    s = lax.axis_index("s")
    wid = c * NS + s
    row0 = wid * NCHUNK
    pltpu.sync_copy(src2_hbm.at[pl.ds(row0, NCHUNK)], sidx_v)
    pltpu.sync_copy(dst2_hbm.at[pl.ds(row0, NCHUNK)], didx_v)
    pltpu.sync_copy(zeros_hbm, acc_sh.at[pl.ds(s * RPS, RPS)])
    plsc.subcore_barrier()

    pltpu.async_copy(h_hbm.at[sidx_v.at[0]], rows0_v, gsem0)

    @pl.loop(0, NCHUNK // 2)
    def _(g2):
        a = 2 * g2
        pltpu.async_copy(h_hbm.at[sidx_v.at[a + 1]], rows1_v, gsem1)
        pltpu.make_async_copy(h_hbm.at[sidx_v.at[a]], rows0_v, gsem0).wait()
        pltpu.sync_copy(rows0_v, acc_sh.at[didx_v.at[a]], add=True)
        nxt = jnp.minimum(a + 2, NCHUNK - 1)
        pltpu.async_copy(h_hbm.at[sidx_v.at[nxt]], rows0_v, gsem0)
        pltpu.make_async_copy(h_hbm.at[sidx_v.at[a + 1]], rows1_v, gsem1).wait()
        pltpu.sync_copy(rows1_v, acc_sh.at[didx_v.at[a + 1]], add=True)

    pltpu.make_async_copy(h_hbm.at[sidx_v.at[0]], rows0_v, gsem0).wait()

    plsc.subcore_barrier()
    rows = pl.ds(s * RPS, RPS)
    pltpu.sync_copy(acc_sh.at[rows], out_hbm.at[c, rows])


def _seg_sum(h, src2, dst2):
    zeros = jnp.zeros((RPS, D), jnp.float32)
    f = pl.kernel(
        _seg_body,
        out_type=jax.ShapeDtypeStruct((NC, NP, D), jnp.float32),
        mesh=_mesh,
        scratch_types=[
            pltpu.VMEM((NCHUNK, CHUNK), jnp.int32),
            pltpu.VMEM((NCHUNK, CHUNK), jnp.int32),
            pltpu.VMEM((CHUNK, D), jnp.float32),
            pltpu.VMEM((CHUNK, D), jnp.float32),
            pltpu.VMEM_SHARED((NP, D), jnp.float32),
            pltpu.SemaphoreType.DMA,
            pltpu.SemaphoreType.DMA,
        ],
        compiler_params=_sc_params,
    )
    return f(h, src2, dst2, zeros)


def _mm_body(x_ref, w_ref, o_ref):
    o_ref[...] = jnp.dot(x_ref[...], w_ref[...],
                         preferred_element_type=jnp.float32,
                         precision=lax.Precision.HIGHEST)


def _matmul(x, w):
    return pl.pallas_call(
        _mm_body,
        out_shape=jax.ShapeDtypeStruct((N, D), jnp.float32),
    )(x, w)


def _norm_scale_body(scnt_ref, dcnt_ref, h_ref, h_out, onorm_out, inorm_out):
    sdeg = scnt_ref[0, :N, 0:1] + scnt_ref[1, :N, 0:1]
    ddeg = dcnt_ref[0, :N, 0:1] + dcnt_ref[1, :N, 0:1]
    onorm = lax.rsqrt(jnp.maximum(sdeg, 1.0))
    inorm = lax.rsqrt(jnp.maximum(ddeg, 1.0))
    onorm_out[...] = onorm
    inorm_out[...] = inorm
    h_out[...] = h_ref[...] * onorm


def _norm_scale(scnt, dcnt, h_raw):
    return pl.pallas_call(
        _norm_scale_body,
        out_shape=[jax.ShapeDtypeStruct((N, D), jnp.float32),
                   jax.ShapeDtypeStruct((N, 1), jnp.float32),
                   jax.ShapeDtypeStruct((N, 1), jnp.float32)],
    )(scnt, dcnt, h_raw)


def _mid_body(p_ref, inorm_ref, onorm_ref, b1_ref, w2_ref, h2_out):
    x = (p_ref[0, :N] + p_ref[1, :N]) * inorm_ref[...] + b1_ref[...]
    x = jnp.maximum(x, 0.0) * onorm_ref[...]
    h2_out[...] = jnp.dot(x, w2_ref[...],
                          preferred_element_type=jnp.float32,
                          precision=lax.Precision.HIGHEST)


def _mid(p, inorm, onorm, b1, w2):
    return pl.pallas_call(
        _mid_body,
        out_shape=jax.ShapeDtypeStruct((N, D), jnp.float32),
    )(p, inorm, onorm, b1, w2)


def _final_body(q_ref, inorm_ref, b2_ref, o_ref):
    o_ref[...] = (q_ref[0, :N] + q_ref[1, :N]) * inorm_ref[...] + b2_ref[...]


def _final(q, inorm, b2):
    return pl.pallas_call(
        _final_body,
        out_shape=jax.ShapeDtypeStruct((N, D), jnp.float32),
    )(q, inorm, b2)


def kernel(features, edge_index, W1, b1, W2, b2):
    src2 = edge_index[0].reshape(E // CHUNK, CHUNK)
    dst2 = edge_index[1].reshape(E // CHUNK, CHUNK)
    scnt, dcnt = _degrees(src2, dst2)
    h1_raw = _matmul(features, W1)
    h1, onorm, inorm = _norm_scale(scnt, dcnt, h1_raw)
    p = _seg_sum(h1, src2, dst2)
    h2 = _mid(p, inorm, onorm, b1.reshape(1, D), W2)
    q = _seg_sum(h2, src2, dst2)
    return _final(q, inorm, b2.reshape(1, D))


# speedup vs baseline: 10.8268x; 1.0953x over previous
import functools

import jax
import jax.numpy as jnp
from jax import lax
from jax.experimental import pallas as pl
from jax.experimental.pallas import tpu as pltpu
from jax.experimental.pallas import tpu_sc as plsc

N = 10000
E = 320000
D = 128

NC = 2
NS = 16
NW = NC * NS
EPT = E // NW
NP = 10240
RPS = NP // NS
CHUNK = 128
NFULL = EPT // CHUNK
TAIL = EPT - NFULL * CHUNK
CW = 16

_mesh = plsc.VectorSubcoreMesh(core_axis_name="c", subcore_axis_name="s")
_sc_params = pltpu.CompilerParams(use_tc_tiling_on_sc=False)


def _degree_body(ei_hbm, zeros_hbm, ones_hbm,
                 scnt_hbm, dcnt_hbm, sidx_v, didx_v, ones_v,
                 scnt_sh, dcnt_sh, ssem, dsem):
    c = lax.axis_index("c")
    s = lax.axis_index("s")
    wid = c * NS + s
    base0 = wid * EPT
    pltpu.sync_copy(ones_hbm, ones_v)
    pltpu.sync_copy(ei_hbm.at[0, pl.ds(base0, EPT)], sidx_v)
    pltpu.sync_copy(ei_hbm.at[1, pl.ds(base0, EPT)], didx_v)
    pltpu.sync_copy(zeros_hbm, scnt_sh.at[pl.ds(s * RPS, RPS)])
    pltpu.sync_copy(zeros_hbm, dcnt_sh.at[pl.ds(s * RPS, RPS)])
    plsc.subcore_barrier()

    @pl.loop(0, NFULL)
    def _(i):
        e = i * CHUNK
        a = pltpu.async_copy(
            ones_v, scnt_sh.at[sidx_v.at[pl.ds(e, CHUNK)]], ssem, add=True)
        b = pltpu.async_copy(
            ones_v, dcnt_sh.at[didx_v.at[pl.ds(e, CHUNK)]], dsem, add=True)
        a.wait()
        b.wait()

    e = NFULL * CHUNK
    a = pltpu.async_copy(
        ones_v.at[pl.ds(0, TAIL)], scnt_sh.at[sidx_v.at[pl.ds(e, TAIL)]],
        ssem, add=True)
    b = pltpu.async_copy(
        ones_v.at[pl.ds(0, TAIL)], dcnt_sh.at[didx_v.at[pl.ds(e, TAIL)]],
        dsem, add=True)
    a.wait()
    b.wait()

    plsc.subcore_barrier()
    rows = pl.ds(s * RPS, RPS)
    pltpu.sync_copy(scnt_sh.at[rows], scnt_hbm.at[c, rows])
    pltpu.sync_copy(dcnt_sh.at[rows], dcnt_hbm.at[c, rows])


def _degrees(edge_index):
    zeros = jnp.zeros((RPS, CW), jnp.float32)
    ones = jnp.ones((CHUNK, CW), jnp.float32)
    f = pl.kernel(
        _degree_body,
        out_type=(jax.ShapeDtypeStruct((NC, NP, CW), jnp.float32),
                  jax.ShapeDtypeStruct((NC, NP, CW), jnp.float32)),
        mesh=_mesh,
        scratch_types=[
            pltpu.VMEM((EPT,), jnp.int32),
            pltpu.VMEM((EPT,), jnp.int32),
            pltpu.VMEM((CHUNK, CW), jnp.float32),
            pltpu.VMEM_SHARED((NP, CW), jnp.float32),
            pltpu.VMEM_SHARED((NP, CW), jnp.float32),
            pltpu.SemaphoreType.DMA,
            pltpu.SemaphoreType.DMA,
        ],
        compiler_params=_sc_params,
    )
    return f(edge_index, zeros, ones)


def _seg_body(h_hbm, ei_hbm, zeros_hbm, out_hbm,
              sidx_v, db0_v, db1_v, dt_v, rows0_v, rows1_v, acc_sh,
              gsem0, gsem1, dsem0, dsem1):
    c = lax.axis_index("c")
    s = lax.axis_index("s")
    wid = c * NS + s
    base0 = wid * EPT

    def sidx(i):
        return sidx_v.at[pl.ds(i * CHUNK, CHUNK)]

    def didx(i):
        return ei_hbm.at[1, pl.ds(base0 + i * CHUNK, CHUNK)]

    pltpu.sync_copy(ei_hbm.at[0, pl.ds(base0, EPT)], sidx_v)
    pltpu.sync_copy(zeros_hbm, acc_sh.at[pl.ds(s * RPS, RPS)])
    plsc.subcore_barrier()

    pltpu.async_copy(didx(0), db0_v, dsem0)
    pltpu.async_copy(h_hbm.at[sidx(0)], rows0_v, gsem0)

    @pl.loop(0, NFULL // 2)
    def _(g2):
        a = 2 * g2
        pltpu.async_copy(didx(a + 1), db1_v, dsem1)
        pltpu.async_copy(h_hbm.at[sidx(a + 1)], rows1_v, gsem1)
        pltpu.make_async_copy(didx(0), db0_v, dsem0).wait()
        pltpu.make_async_copy(h_hbm.at[sidx(0)], rows0_v, gsem0).wait()
        pltpu.sync_copy(rows0_v, acc_sh.at[db0_v], add=True)
        nxt = jnp.minimum(a + 2, NFULL - 1)
        pltpu.async_copy(didx(nxt), db0_v, dsem0)
        pltpu.async_copy(h_hbm.at[sidx(nxt)], rows0_v, gsem0)
        pltpu.make_async_copy(didx(0), db1_v, dsem1).wait()
        pltpu.make_async_copy(h_hbm.at[sidx(0)], rows1_v, gsem1).wait()
        pltpu.sync_copy(rows1_v, acc_sh.at[db1_v], add=True)

    pltpu.make_async_copy(didx(0), db0_v, dsem0).wait()
    pltpu.make_async_copy(h_hbm.at[sidx(0)], rows0_v, gsem0).wait()
    e = NFULL * CHUNK
    pltpu.sync_copy(ei_hbm.at[1, pl.ds(base0 + e, TAIL)], dt_v)
    pltpu.sync_copy(h_hbm.at[sidx_v.at[pl.ds(e, TAIL)]],
                    rows0_v.at[pl.ds(0, TAIL)])
    pltpu.sync_copy(rows0_v.at[pl.ds(0, TAIL)], acc_sh.at[dt_v], add=True)

    plsc.subcore_barrier()
    rows = pl.ds(s * RPS, RPS)
    pltpu.sync_copy(acc_sh.at[rows], out_hbm.at[c, rows])


def _seg_sum(h, edge_index):
    zeros = jnp.zeros((RPS, D), jnp.float32)
    f = pl.kernel(
        _seg_body,
        out_type=jax.ShapeDtypeStruct((NC, NP, D), jnp.float32),
        mesh=_mesh,
        scratch_types=[
            pltpu.VMEM((EPT,), jnp.int32),
            pltpu.VMEM((CHUNK,), jnp.int32),
            pltpu.VMEM((CHUNK,), jnp.int32),
            pltpu.VMEM((TAIL,), jnp.int32),
            pltpu.VMEM((CHUNK, D), jnp.float32),
            pltpu.VMEM((CHUNK, D), jnp.float32),
            pltpu.VMEM_SHARED((NP, D), jnp.float32),
            pltpu.SemaphoreType.DMA,
            pltpu.SemaphoreType.DMA,
            pltpu.SemaphoreType.DMA,
            pltpu.SemaphoreType.DMA,
        ],
        compiler_params=_sc_params,
    )
    return f(h, edge_index, zeros)


def _mm_body(x_ref, w_ref, o_ref):
    o_ref[...] = jnp.dot(x_ref[...], w_ref[...],
                         preferred_element_type=jnp.float32,
                         precision=lax.Precision.HIGHEST)


def _matmul(x, w):
    return pl.pallas_call(
        _mm_body,
        out_shape=jax.ShapeDtypeStruct((N, D), jnp.float32),
    )(x, w)


def _norm_scale_body(scnt_ref, dcnt_ref, h_ref, h_out, onorm_out, inorm_out):
    sdeg = scnt_ref[0, :N, 0:1] + scnt_ref[1, :N, 0:1]
    ddeg = dcnt_ref[0, :N, 0:1] + dcnt_ref[1, :N, 0:1]
    onorm = lax.rsqrt(jnp.maximum(sdeg, 1.0))
    inorm = lax.rsqrt(jnp.maximum(ddeg, 1.0))
    onorm_out[...] = onorm
    inorm_out[...] = inorm
    h_out[...] = h_ref[...] * onorm


def _norm_scale(scnt, dcnt, h_raw):
    return pl.pallas_call(
        _norm_scale_body,
        out_shape=[jax.ShapeDtypeStruct((N, D), jnp.float32),
                   jax.ShapeDtypeStruct((N, 1), jnp.float32),
                   jax.ShapeDtypeStruct((N, 1), jnp.float32)],
    )(scnt, dcnt, h_raw)


def _mid_body(p_ref, inorm_ref, onorm_ref, b1_ref, w2_ref, h2_out):
    x = (p_ref[0, :N] + p_ref[1, :N]) * inorm_ref[...] + b1_ref[...]
    x = jnp.maximum(x, 0.0) * onorm_ref[...]
    h2_out[...] = jnp.dot(x, w2_ref[...],
                          preferred_element_type=jnp.float32,
                          precision=lax.Precision.HIGHEST)


def _mid(p, inorm, onorm, b1, w2):
    return pl.pallas_call(
        _mid_body,
        out_shape=jax.ShapeDtypeStruct((N, D), jnp.float32),
    )(p, inorm, onorm, b1, w2)


def _final_body(q_ref, inorm_ref, b2_ref, o_ref):
    o_ref[...] = (q_ref[0, :N] + q_ref[1, :N]) * inorm_ref[...] + b2_ref[...]


def _final(q, inorm, b2):
    return pl.pallas_call(
        _final_body,
        out_shape=jax.ShapeDtypeStruct((N, D), jnp.float32),
    )(q, inorm, b2)


def kernel(features, edge_index, W1, b1, W2, b2):
    scnt, dcnt = _degrees(edge_index)
    h1_raw = _matmul(features, W1)
    h1, onorm, inorm = _norm_scale(scnt, dcnt, h1_raw)
    p = _seg_sum(h1, edge_index)
    h2 = _mid(p, inorm, onorm, b1.reshape(1, D), W2)
    q = _seg_sum(h2, edge_index)
    return _final(q, inorm, b2.reshape(1, D))


# speedup vs baseline: 10.8724x; 1.0042x over previous
import functools

import jax
import jax.numpy as jnp
from jax import lax
from jax.experimental import pallas as pl
from jax.experimental.pallas import tpu as pltpu
from jax.experimental.pallas import tpu_sc as plsc

N = 10000
E = 320000
D = 128

NC = 2
NS = 16
NW = NC * NS
EPT = E // NW
NP = 10240
RPS = NP // NS
CHUNK = 128
NFULL = EPT // CHUNK
TAIL = EPT - NFULL * CHUNK
CW = 16

_mesh = plsc.VectorSubcoreMesh(core_axis_name="c", subcore_axis_name="s")
_sc_params = pltpu.CompilerParams(use_tc_tiling_on_sc=False)


def _degree_body(ei_hbm, zeros_hbm, ones_hbm,
                 scnt_hbm, dcnt_hbm, sidx_v, didx_v, ones_v,
                 scnt_sh, dcnt_sh, ssem, dsem):
    c = lax.axis_index("c")
    s = lax.axis_index("s")
    wid = c * NS + s
    base0 = wid * EPT
    pltpu.sync_copy(ones_hbm, ones_v)
    pltpu.sync_copy(ei_hbm.at[0, pl.ds(base0, EPT)], sidx_v)
    pltpu.sync_copy(ei_hbm.at[1, pl.ds(base0, EPT)], didx_v)
    pltpu.sync_copy(zeros_hbm, scnt_sh.at[pl.ds(s * RPS, RPS)])
    pltpu.sync_copy(zeros_hbm, dcnt_sh.at[pl.ds(s * RPS, RPS)])
    plsc.subcore_barrier()

    @pl.loop(0, NFULL)
    def _(i):
        e = i * CHUNK
        a = pltpu.async_copy(
            ones_v, scnt_sh.at[sidx_v.at[pl.ds(e, CHUNK)]], ssem, add=True)
        b = pltpu.async_copy(
            ones_v, dcnt_sh.at[didx_v.at[pl.ds(e, CHUNK)]], dsem, add=True)
        a.wait()
        b.wait()

    e = NFULL * CHUNK
    a = pltpu.async_copy(
        ones_v.at[pl.ds(0, TAIL)], scnt_sh.at[sidx_v.at[pl.ds(e, TAIL)]],
        ssem, add=True)
    b = pltpu.async_copy(
        ones_v.at[pl.ds(0, TAIL)], dcnt_sh.at[didx_v.at[pl.ds(e, TAIL)]],
        dsem, add=True)
    a.wait()
    b.wait()

    plsc.subcore_barrier()
    rows = pl.ds(s * RPS, RPS)
    pltpu.sync_copy(scnt_sh.at[rows], scnt_hbm.at[c, rows])
    pltpu.sync_copy(dcnt_sh.at[rows], dcnt_hbm.at[c, rows])


def _degrees(edge_index):
    zeros = jnp.zeros((RPS, CW), jnp.float32)
    ones = jnp.ones((CHUNK, CW), jnp.float32)
    f = pl.kernel(
        _degree_body,
        out_type=(jax.ShapeDtypeStruct((NC, NP, CW), jnp.float32),
                  jax.ShapeDtypeStruct((NC, NP, CW), jnp.float32)),
        mesh=_mesh,
        scratch_types=[
            pltpu.VMEM((EPT,), jnp.int32),
            pltpu.VMEM((EPT,), jnp.int32),
            pltpu.VMEM((CHUNK, CW), jnp.float32),
            pltpu.VMEM_SHARED((NP, CW), jnp.float32),
            pltpu.VMEM_SHARED((NP, CW), jnp.float32),
            pltpu.SemaphoreType.DMA,
            pltpu.SemaphoreType.DMA,
        ],
        compiler_params=_sc_params,
    )
    return f(edge_index, zeros, ones)


def _seg_body(h_hbm, ei_hbm, zeros_hbm, out_hbm,
              sidx_v, db0_v, db1_v, dt_v, rows0_v, rows1_v, acc_sh,
              gsem0, gsem1, dsem0, dsem1):
    c = lax.axis_index("c")
    s = lax.axis_index("s")
    wid = c * NS + s
    base0 = wid * EPT

    def sidx(i):
        return sidx_v.at[pl.ds(i * CHUNK, CHUNK)]

    def didx(i):
        return ei_hbm.at[1, pl.ds(base0 + i * CHUNK, CHUNK)]

    pltpu.sync_copy(ei_hbm.at[0, pl.ds(base0, EPT)], sidx_v)
    pltpu.sync_copy(zeros_hbm, acc_sh.at[pl.ds(s * RPS, RPS)])
    plsc.subcore_barrier()

    pltpu.async_copy(didx(0), db0_v, dsem0)
    pltpu.async_copy(h_hbm.at[sidx(0)], rows0_v, gsem0)

    @pl.loop(0, NFULL // 2)
    def _(g2):
        a = 2 * g2
        pltpu.async_copy(didx(a + 1), db1_v, dsem1)
        pltpu.async_copy(h_hbm.at[sidx(a + 1)], rows1_v, gsem1)
        pltpu.make_async_copy(didx(0), db0_v, dsem0).wait()
        pltpu.make_async_copy(h_hbm.at[sidx(0)], rows0_v, gsem0).wait()
        pltpu.sync_copy(rows0_v, acc_sh.at[db0_v], add=True)
        nxt = jnp.minimum(a + 2, NFULL - 1)
        pltpu.async_copy(didx(nxt), db0_v, dsem0)
        pltpu.async_copy(h_hbm.at[sidx(nxt)], rows0_v, gsem0)
        pltpu.make_async_copy(didx(0), db1_v, dsem1).wait()
        pltpu.make_async_copy(h_hbm.at[sidx(0)], rows1_v, gsem1).wait()
        pltpu.sync_copy(rows1_v, acc_sh.at[db1_v], add=True)

    pltpu.make_async_copy(didx(0), db0_v, dsem0).wait()
    pltpu.make_async_copy(h_hbm.at[sidx(0)], rows0_v, gsem0).wait()
    e = NFULL * CHUNK
    pltpu.sync_copy(ei_hbm.at[1, pl.ds(base0 + e, TAIL)], dt_v)
    pltpu.sync_copy(h_hbm.at[sidx_v.at[pl.ds(e, TAIL)]],
                    rows0_v.at[pl.ds(0, TAIL)])
    pltpu.sync_copy(rows0_v.at[pl.ds(0, TAIL)], acc_sh.at[dt_v], add=True)

    plsc.subcore_barrier()
    rows = pl.ds(s * RPS, RPS)
    pltpu.sync_copy(acc_sh.at[rows], out_hbm.at[c, rows])


def _seg_sum(h, edge_index):
    zeros = jnp.zeros((RPS, D), jnp.float32)
    f = pl.kernel(
        _seg_body,
        out_type=jax.ShapeDtypeStruct((NC, NP, D), jnp.float32),
        mesh=_mesh,
        scratch_types=[
            pltpu.VMEM((EPT,), jnp.int32),
            pltpu.VMEM((CHUNK,), jnp.int32),
            pltpu.VMEM((CHUNK,), jnp.int32),
            pltpu.VMEM((TAIL,), jnp.int32),
            pltpu.VMEM((CHUNK, D), jnp.float32),
            pltpu.VMEM((CHUNK, D), jnp.float32),
            pltpu.VMEM_SHARED((NP, D), jnp.float32),
            pltpu.SemaphoreType.DMA,
            pltpu.SemaphoreType.DMA,
            pltpu.SemaphoreType.DMA,
            pltpu.SemaphoreType.DMA,
        ],
        compiler_params=_sc_params,
    )
    return f(h, edge_index, zeros)


RB = 2000
GRID = N // RB


def _mm_body(x_ref, w_ref, o_ref):
    o_ref[...] = jnp.dot(x_ref[...], w_ref[...],
                         preferred_element_type=jnp.float32,
                         precision=lax.Precision.HIGHEST)


def _matmul(x, w):
    return pl.pallas_call(
        _mm_body,
        grid=(GRID,),
        in_specs=[pl.BlockSpec((RB, D), lambda i: (i, 0)),
                  pl.BlockSpec((D, D), lambda i: (0, 0))],
        out_specs=pl.BlockSpec((RB, D), lambda i: (i, 0)),
        out_shape=jax.ShapeDtypeStruct((N, D), jnp.float32),
    )(x, w)


def _norm_scale_body(scnt_ref, dcnt_ref, h_ref, h_out, onorm_out, inorm_out,
                     cnt_v, sem):
    i = pl.program_id(0)

    @pl.when(i == 0)
    def _():
        pltpu.make_async_copy(scnt_ref, cnt_v.at[0], sem).start()
        pltpu.make_async_copy(dcnt_ref, cnt_v.at[1], sem).start()
        pltpu.make_async_copy(scnt_ref, cnt_v.at[0], sem).wait()
        pltpu.make_async_copy(dcnt_ref, cnt_v.at[1], sem).wait()

    r = pl.ds(i * RB, RB)
    sdeg = cnt_v[0, 0, r, 0:1] + cnt_v[0, 1, r, 0:1]
    ddeg = cnt_v[1, 0, r, 0:1] + cnt_v[1, 1, r, 0:1]
    onorm = lax.rsqrt(jnp.maximum(sdeg, 1.0))
    inorm = lax.rsqrt(jnp.maximum(ddeg, 1.0))
    onorm_out[...] = onorm
    inorm_out[...] = inorm
    h_out[...] = h_ref[...] * onorm


def _norm_scale(scnt, dcnt, h_raw):
    return pl.pallas_call(
        _norm_scale_body,
        grid=(GRID,),
        in_specs=[pl.BlockSpec(memory_space=pl.ANY),
                  pl.BlockSpec(memory_space=pl.ANY),
                  pl.BlockSpec((RB, D), lambda i: (i, 0))],
        out_specs=[pl.BlockSpec((RB, D), lambda i: (i, 0)),
                   pl.BlockSpec((RB, 1), lambda i: (i, 0)),
                   pl.BlockSpec((RB, 1), lambda i: (i, 0))],
        out_shape=[jax.ShapeDtypeStruct((N, D), jnp.float32),
                   jax.ShapeDtypeStruct((N, 1), jnp.float32),
                   jax.ShapeDtypeStruct((N, 1), jnp.float32)],
        scratch_shapes=[pltpu.VMEM((2, NC, NP, CW), jnp.float32),
                        pltpu.SemaphoreType.DMA],
    )(scnt, dcnt, h_raw)


def _mid_body(p_ref, inorm_ref, onorm_ref, b1_ref, w2_ref, h2_out):
    x = (p_ref[0] + p_ref[1]) * inorm_ref[...] + b1_ref[...]
    x = jnp.maximum(x, 0.0) * onorm_ref[...]
    h2_out[...] = jnp.dot(x, w2_ref[...],
                          preferred_element_type=jnp.float32,
                          precision=lax.Precision.HIGHEST)


def _mid(p, inorm, onorm, b1, w2):
    return pl.pallas_call(
        _mid_body,
        grid=(GRID,),
        in_specs=[pl.BlockSpec((NC, RB, D), lambda i: (0, i, 0)),
                  pl.BlockSpec((RB, 1), lambda i: (i, 0)),
                  pl.BlockSpec((RB, 1), lambda i: (i, 0)),
                  pl.BlockSpec((1, D), lambda i: (0, 0)),
                  pl.BlockSpec((D, D), lambda i: (0, 0))],
        out_specs=pl.BlockSpec((RB, D), lambda i: (i, 0)),
        out_shape=jax.ShapeDtypeStruct((N, D), jnp.float32),
    )(p, inorm, onorm, b1, w2)


def _final_body(q_ref, inorm_ref, b2_ref, o_ref):
    o_ref[...] = (q_ref[0] + q_ref[1]) * inorm_ref[...] + b2_ref[...]


def _final(q, inorm, b2):
    return pl.pallas_call(
        _final_body,
        grid=(GRID,),
        in_specs=[pl.BlockSpec((NC, RB, D), lambda i: (0, i, 0)),
                  pl.BlockSpec((RB, 1), lambda i: (i, 0)),
                  pl.BlockSpec((1, D), lambda i: (0, 0))],
        out_specs=pl.BlockSpec((RB, D), lambda i: (i, 0)),
        out_shape=jax.ShapeDtypeStruct((N, D), jnp.float32),
    )(q, inorm, b2)


def kernel(features, edge_index, W1, b1, W2, b2):
    scnt, dcnt = _degrees(edge_index)
    h1_raw = _matmul(features, W1)
    h1, onorm, inorm = _norm_scale(scnt, dcnt, h1_raw)
    p = _seg_sum(h1, edge_index)
    h2 = _mid(p, inorm, onorm, b1.reshape(1, D), W2)
    q = _seg_sum(h2, edge_index)
    return _final(q, inorm, b2.reshape(1, D))
